# Initial kernel scaffold; baseline (speedup 1.0000x reference)
import jax
import jax.numpy as jnp
from jax.experimental import pallas as pl


def kernel(exer_emb, kn_emb, ei_directed, ei_undirected, ei_ek, W_dir, A_dir, W_und, A_und, W_ek, A_ek, Wk1, bk1, Wk2, bk2, We1, be1):
    raise NotImplementedError("write your pallas kernel here")


# speedup vs baseline: 4.8097x; 4.8097x over previous
import functools

import jax
import jax.numpy as jnp
from jax import lax
from jax.experimental import pallas as pl
from jax.experimental.pallas import tpu as pltpu
from jax.experimental.pallas import tpu_sc as plsc

NK = 10000
NE = 50000
D = 128
CH = 12504
TRASH0 = CH
SROW = 12520
SNR = 104
TBLP = 12672
BLK = 32
NSUB = 16
E_G = 320000
E_EK = 400000
NB_G = E_G // BLK
NB_EK = E_EK // BLK

_f32 = jnp.float32


def _prep_body(kn_ref, w_ref, a_ref, y_ref, u_ref):
    kn = kn_ref[...]
    w = w_ref[...]
    z = jnp.dot(kn, w.T, preferred_element_type=_f32,
                precision=lax.Precision.HIGHEST)
    a1 = a_ref[...][:, :D]
    p = jnp.dot(z, a1.T, preferred_element_type=_f32,
                precision=lax.Precision.HIGHEST)
    u = jnp.exp(p - jnp.max(p))
    y_ref[...] = z * u
    u_ref[...] = u


_prep_one = pl.pallas_call(
    _prep_body,
    out_shape=[jax.ShapeDtypeStruct((NK, D), _f32),
               jax.ShapeDtypeStruct((NK, 1), _f32)],
)


def _prep(kn, wd, wu, we, ad, au, ae):
    yd, ud = _prep_one(kn, wd, ad)
    yu, uu = _prep_one(kn, wu, au)
    ye, ue = _prep_one(kn, we, ae)
    return yd, yu, ye, ud, uu, ue


@functools.lru_cache(maxsize=1)
def _build_sc_gat():
    mesh = plsc.VectorSubcoreMesh(core_axis_name="c", subcore_axis_name="s")

    @functools.partial(
        pl.kernel,
        out_type=[
            jax.ShapeDtypeStruct((NK, D), _f32),
            jax.ShapeDtypeStruct((NK, D), _f32),
            jax.ShapeDtypeStruct((4 * CH, D), _f32),
            jax.ShapeDtypeStruct((SNR, D), _f32),
            jax.ShapeDtypeStruct((SNR, D), _f32),
            jax.ShapeDtypeStruct((4 * SNR, D), _f32),
        ],
        mesh=mesh,
        compiler_params=pltpu.CompilerParams(needs_layout_passes=False),
        scratch_types=[
            pltpu.VMEM_SHARED((TBLP, D), _f32),
            pltpu.VMEM((BLK,), jnp.int32),
            pltpu.VMEM((BLK,), jnp.int32),
            pltpu.VMEM((BLK, D), _f32),
            pltpu.VMEM((NK,), _f32),
            pltpu.VMEM((SNR, D), _f32),
            pltpu.VMEM((SNR,), jnp.int32),
        ],
    )
    def sc_gat(yd, yu, ye, ud, uu, ue, sd, dd, su, du, sek, dek, zz,
               od, ou, oe, osd, osu, ose,
               acc, isrc, idst, rows, uv_t, s_loc, sidx):
        sid = lax.axis_index("s")
        cid = lax.axis_index("c")

        for j in range(SNR // 16):
            sidx[pl.ds(j * 16, 16)] = lax.iota(jnp.int32, 16) + (SROW + j * 16)
        if SNR % 16:
            o = SNR - 16
            sidx[pl.ds(o, 16)] = lax.iota(jnp.int32, 16) + (SROW + o)

        def run_pass(y_hbm, u_hbm, s_hbm, d_hbm, nblocks, base, chunked,
                     out_hbm, out_off, out_rows, outs_hbm, outs_off):
            zshare = TBLP // NSUB
            z0 = sid * zshare
            pltpu.sync_copy(zz.at[pl.ds(z0, zshare)],
                            acc.at[pl.ds(z0, zshare)])
            pltpu.sync_copy(zz.at[pl.ds(0, SNR)], s_loc)
            pltpu.sync_copy(u_hbm, uv_t)
            plsc.subcore_barrier()

            nloop = -(-nblocks // NSUB)

            @pl.loop(0, nloop)
            def _(i):
                b = i * NSUB + sid

                @pl.when(b < nblocks)
                def _():
                    off = b * BLK
                    pltpu.sync_copy(s_hbm.at[pl.ds(off, BLK)], isrc)
                    pltpu.sync_copy(d_hbm.at[pl.ds(off, BLK)], idst)
                    for k in range(BLK // 16):
                        sl = pl.ds(k * 16, 16)
                        d16 = idst[sl]
                        s16 = isrc[sl]
                        uv = plsc.load_gather(uv_t, [s16])
                        if chunked:
                            t = d16 - base
                            ok = (t >= 0) & (t < CH)
                            tc = jnp.where(ok, t, 0)
                            idst[sl] = jnp.where(ok, t, TRASH0 + (d16 & 15))
                            plsc.addupdate_scatter(
                                s_loc,
                                [lax.shift_right_logical(tc, 7), tc & 127],
                                uv, mask=ok)
                        else:
                            plsc.addupdate_scatter(
                                s_loc,
                                [lax.shift_right_logical(d16, 7), d16 & 127],
                                uv)
                    pltpu.sync_copy(y_hbm.at[isrc], rows)
                    pltpu.sync_copy(rows, acc.at[idst], add=True)

            pltpu.sync_copy(s_loc, acc.at[sidx], add=True)
            plsc.subcore_barrier()

            lo = (out_rows // NSUB) & ~7
            rem = out_rows - (NSUB - 1) * lo

            @pl.when(sid < NSUB - 1)
            def _():
                o0 = sid * lo
                pltpu.sync_copy(acc.at[pl.ds(o0, lo)],
                                out_hbm.at[pl.ds(out_off + o0, lo)])

            @pl.when(sid == NSUB - 1)
            def _():
                o0 = (NSUB - 1) * lo
                pltpu.sync_copy(acc.at[pl.ds(o0, rem)],
                                out_hbm.at[pl.ds(out_off + o0, rem)])

            @pl.when(sid == 0)
            def _():
                pltpu.sync_copy(acc.at[pl.ds(SROW, SNR)],
                                outs_hbm.at[pl.ds(outs_off, SNR)])

            plsc.subcore_barrier()

        @pl.when(cid == 0)
        def _():
            run_pass(yd, ud, sd, dd, NB_G, 0, False, od, 0, NK, osd, 0)
            run_pass(ye, ue, sek, dek, NB_EK, 0, True, oe, 0, CH, ose, 0)
            run_pass(ye, ue, sek, dek, NB_EK, CH, True,
                     oe, CH, CH, ose, SNR)

        @pl.when(cid == 1)
        def _():
            run_pass(yu, uu, su, du, NB_G, 0, False, ou, 0, NK, osu, 0)
            run_pass(ye, ue, sek, dek, NB_EK, 2 * CH, True,
                     oe, 2 * CH, CH, ose, 2 * SNR)
            run_pass(ye, ue, sek, dek, NB_EK, 3 * CH, True,
                     oe, 3 * CH, CH, ose, 3 * SNR)

    return sc_gat


def _fuse_kn_body(kn_ref, dz_ref, ds_ref, uz_ref, us_ref,
                  w1_ref, b1_ref, w2_ref, b2_ref, out_ref):
    kn = kn_ref[...]
    bk = dz_ref[...] / (ds_ref[...] + 1e-9)
    ck = uz_ref[...] / (us_ref[...] + 1e-9)
    w1 = w1_ref[...]
    w2 = w2_ref[...]
    s1 = (jnp.dot(kn, w1[:, :D].T, preferred_element_type=_f32)
          + jnp.dot(bk, w1[:, D:].T, preferred_element_type=_f32)
          + b1_ref[...])
    s2 = (jnp.dot(kn, w2[:, :D].T, preferred_element_type=_f32)
          + jnp.dot(ck, w2[:, D:].T, preferred_element_type=_f32)
          + b2_ref[...])
    m = jnp.maximum(s1, s2)
    e1 = jnp.exp(s1 - m)
    e2 = jnp.exp(s2 - m)
    out_ref[...] = kn + (e1 * bk + e2 * ck) / (e1 + e2)


_fuse_kn = pl.pallas_call(
    _fuse_kn_body,
    out_shape=jax.ShapeDtypeStruct((NK, D), _f32),
)


def _fuse_ex_body(ex_ref, bz_ref, bs_ref, w_ref, b_ref, out_ref):
    ex = ex_ref[...]
    be = bz_ref[...] / (bs_ref[...] + 1e-9)
    w = w_ref[...]
    se = (jnp.dot(ex, w[:, :D].T, preferred_element_type=_f32)
          + jnp.dot(be, w[:, D:].T, preferred_element_type=_f32)
          + b_ref[...])
    out_ref[...] = ex + se * be


_EXB = 10000

_fuse_ex = pl.pallas_call(
    _fuse_ex_body,
    grid=(NE // _EXB,),
    in_specs=[
        pl.BlockSpec((_EXB, D), lambda i: (i, 0)),
        pl.BlockSpec((_EXB, D), lambda i: (i, 0)),
        pl.BlockSpec((_EXB, 1), lambda i: (i, 0)),
        pl.BlockSpec((1, 2 * D), lambda i: (0, 0)),
        pl.BlockSpec((1, 1), lambda i: (0, 0)),
    ],
    out_specs=pl.BlockSpec((_EXB, D), lambda i: (i, 0)),
    out_shape=jax.ShapeDtypeStruct((NE, D), _f32),
)


def kernel(exer_emb, kn_emb, ei_directed, ei_undirected, ei_ek,
           W_dir, A_dir, W_und, A_und, W_ek, A_ek,
           Wk1, bk1, Wk2, bk2, We1, be1):
    yd, yu, ye, ud, uu, ue = _prep(kn_emb, W_dir, W_und, W_ek,
                                   A_dir, A_und, A_ek)
    zz = jnp.zeros((TBLP, D), _f32)

    od, ou, oe, osd, osu, ose = _build_sc_gat()(
        yd, yu, ye, ud.reshape(NK), uu.reshape(NK), ue.reshape(NK),
        ei_directed[0], ei_directed[1],
        ei_undirected[0], ei_undirected[1],
        ei_ek[0] - NE, ei_ek[1], zz)

    s_d = osd.reshape(-1)[:NK, None]
    s_u = osu.reshape(-1)[:NK, None]
    s_e = jnp.concatenate(
        [ose[c * SNR:(c + 1) * SNR].reshape(-1)[:CH] for c in range(4)]
    )[:NE, None]

    kn_out = _fuse_kn(kn_emb, od, s_d, ou, s_u,
                      Wk1, bk1.reshape(1, 1), Wk2, bk2.reshape(1, 1))
    ex_out = _fuse_ex(exer_emb, oe[:NE], s_e, We1, be1.reshape(1, 1))
    return ex_out, kn_out


# speedup vs baseline: 10.6190x; 2.2079x over previous
import functools

import jax
import jax.numpy as jnp
from jax import lax
from jax.experimental import pallas as pl
from jax.experimental.pallas import tpu as pltpu
from jax.experimental.pallas import tpu_sc as plsc

NK = 10000
NE = 50000
D = 128
CH = 12504
TRASH0 = CH
TBLP = 12544
BLK = 80
NSUB = 16
E_G = 320000
E_EK = 400000
NB_G = E_G // BLK
NB_EK = E_EK // BLK

SBLK = 160
SRW = 512

_f32 = jnp.float32


def _prep_body(kn_ref, w_ref, a_ref, y_ref, u_ref):
    kn = kn_ref[...]
    w = w_ref[...]
    z = jnp.dot(kn, w.T, preferred_element_type=_f32,
                precision=lax.Precision.HIGHEST)
    a1 = a_ref[...][:, :D]
    p = jnp.dot(z, a1.T, preferred_element_type=_f32,
                precision=lax.Precision.HIGHEST)
    u = jnp.exp(p - jnp.max(p))
    y_ref[...] = z * u
    u_ref[...] = u


_prep_one = pl.pallas_call(
    _prep_body,
    out_shape=[jax.ShapeDtypeStruct((NK, D), _f32),
               jax.ShapeDtypeStruct((NK, 1), _f32)],
)


def _prep(kn, wd, wu, we, ad, au, ae):
    yd, ud = _prep_one(kn, wd, ad)
    yu, uu = _prep_one(kn, wu, au)
    ye, ue = _prep_one(kn, we, ae)
    return yd, yu, ye, ud, uu, ue


@functools.lru_cache(maxsize=1)
def _build_sc_rows():
    mesh = plsc.VectorSubcoreMesh(core_axis_name="c", subcore_axis_name="s")

    @functools.partial(
        pl.kernel,
        out_type=[
            jax.ShapeDtypeStruct((NK, D), _f32),
            jax.ShapeDtypeStruct((NK, D), _f32),
            jax.ShapeDtypeStruct((4 * CH, D), _f32),
        ],
        mesh=mesh,
        compiler_params=pltpu.CompilerParams(needs_layout_passes=False),
        scratch_types=[
            pltpu.VMEM_SHARED((TBLP, D), _f32),
            pltpu.VMEM((BLK,), jnp.int32),
            pltpu.VMEM((BLK,), jnp.int32),
            pltpu.VMEM((BLK, D), _f32),
            pltpu.VMEM((BLK,), jnp.int32),
            pltpu.VMEM((BLK,), jnp.int32),
            pltpu.VMEM((BLK, D), _f32),
            pltpu.SemaphoreType.DMA,
            pltpu.SemaphoreType.DMA,
        ],
    )
    def sc_rows(yd, yu, ye, sd, dd, su, du, sek, dek, zz,
                od, ou, oe,
                acc, isa, ida, rwa, isb, idb, rwb, sma, smb):
        sid = lax.axis_index("s")
        cid = lax.axis_index("c")
        bufs = ((isa, ida, rwa, sma), (isb, idb, rwb, smb))

        def run_pass(y_hbm, s_hbm, d_hbm, nblocks, base, chunked,
                     out_hbm, out_off, out_rows):
            zshare = TBLP // NSUB
            z0 = sid * zshare
            pltpu.sync_copy(zz.at[pl.ds(z0, zshare)],
                            acc.at[pl.ds(z0, zshare)])
            plsc.subcore_barrier()

            nloop = -(-nblocks // NSUB)
            npair = -(-nloop // 2)

            @pl.loop(0, npair)
            def _(j):
                for p in range(2):
                    isx, idx_, rws, sem = bufs[p]
                    b = (2 * j + p) * NSUB + sid

                    @pl.when(b < nblocks)
                    def _():
                        off = b * BLK
                        pltpu.sync_copy(s_hbm.at[pl.ds(off, BLK)], isx)
                        pltpu.sync_copy(d_hbm.at[pl.ds(off, BLK)], idx_)
                        if chunked:
                            for k in range(BLK // 16):
                                sl = pl.ds(k * 16, 16)
                                d16 = idx_[sl]
                                t = d16 - base
                                ok = (t >= 0) & (t < CH)
                                idx_[sl] = jnp.where(
                                    ok, t, TRASH0 + (d16 & 15))
                        pltpu.async_copy(y_hbm.at[isx], rws, sem)

                for p in range(2):
                    isx, idx_, rws, sem = bufs[p]
                    b = (2 * j + p) * NSUB + sid

                    @pl.when(b < nblocks)
                    def _():
                        pltpu.make_async_copy(y_hbm.at[isx], rws, sem).wait()
                        pltpu.sync_copy(rws, acc.at[idx_], add=True)

            plsc.subcore_barrier()
            lo = (out_rows // NSUB) & ~7
            rem = out_rows - (NSUB - 1) * lo

            @pl.when(sid < NSUB - 1)
            def _():
                o0 = sid * lo
                pltpu.sync_copy(acc.at[pl.ds(o0, lo)],
                                out_hbm.at[pl.ds(out_off + o0, lo)])

            @pl.when(sid == NSUB - 1)
            def _():
                o0 = (NSUB - 1) * lo
                pltpu.sync_copy(acc.at[pl.ds(o0, rem)],
                                out_hbm.at[pl.ds(out_off + o0, rem)])

            plsc.subcore_barrier()

        @pl.when(cid == 0)
        def _():
            run_pass(yd, sd, dd, NB_G, 0, False, od, 0, NK)
            run_pass(ye, sek, dek, NB_EK, 0, True, oe, 0, CH)
            run_pass(ye, sek, dek, NB_EK, CH, True, oe, CH, CH)

        @pl.when(cid == 1)
        def _():
            run_pass(yu, su, du, NB_G, 0, False, ou, 0, NK)
            run_pass(ye, sek, dek, NB_EK, 2 * CH, True, oe, 2 * CH, CH)
            run_pass(ye, sek, dek, NB_EK, 3 * CH, True, oe, 3 * CH, CH)

    return sc_rows


@functools.lru_cache(maxsize=1)
def _build_sc_sums():
    mesh = plsc.VectorSubcoreMesh(core_axis_name="c", subcore_axis_name="s")

    @functools.partial(
        pl.kernel,
        out_type=[
            jax.ShapeDtypeStruct((SRW, D), _f32),
            jax.ShapeDtypeStruct((SRW, D), _f32),
            jax.ShapeDtypeStruct((SRW, D), _f32),
            jax.ShapeDtypeStruct((SRW, D), _f32),
        ],
        mesh=mesh,
        compiler_params=pltpu.CompilerParams(needs_layout_passes=False),
        scratch_types=[
            pltpu.VMEM_SHARED((SRW, D), _f32),
            pltpu.VMEM((SRW, D), _f32),
            pltpu.VMEM((NK,), _f32),
            pltpu.VMEM((SBLK,), jnp.int32),
            pltpu.VMEM((SBLK,), jnp.int32),
            pltpu.VMEM((128,), jnp.int32),
        ],
    )
    def sc_sums(ud, uu, ue, sd, dd, su, du, sek, dek, zz,
                osd, osu, osea, oseb,
                s_sh, s_loc, uv_t, isx, idx_, sidx):
        sid = lax.axis_index("s")
        cid = lax.axis_index("c")

        def run_pass(u_hbm, s_hbm, d_hbm, e_lo, e_hi, out_hbm):
            pltpu.sync_copy(zz.at[pl.ds(0, SRW)], s_loc)
            share = SRW // NSUB
            z0 = sid * share
            pltpu.sync_copy(zz.at[pl.ds(0, share)],
                            s_sh.at[pl.ds(z0, share)])
            pltpu.sync_copy(u_hbm, uv_t)
            plsc.subcore_barrier()

            nblocks = (e_hi - e_lo) // SBLK
            nloop = -(-nblocks // NSUB)

            @pl.loop(0, nloop)
            def _(i):
                b = i * NSUB + sid

                @pl.when(b < nblocks)
                def _():
                    off = e_lo + b * SBLK
                    pltpu.sync_copy(s_hbm.at[pl.ds(off, SBLK)], isx)
                    pltpu.sync_copy(d_hbm.at[pl.ds(off, SBLK)], idx_)
                    for k in range(SBLK // 16):
                        sl = pl.ds(k * 16, 16)
                        d16 = idx_[sl]
                        uv = plsc.load_gather(uv_t, [isx[sl]])
                        plsc.addupdate_scatter(
                            s_loc,
                            [lax.shift_right_logical(d16, 7), d16 & 127],
                            uv)

            for seg in range(SRW // 128):
                for jj in range(8):
                    sidx[pl.ds(jj * 16, 16)] = (
                        lax.iota(jnp.int32, 16) + (seg * 128 + jj * 16))
                pltpu.sync_copy(s_loc.at[pl.ds(seg * 128, 128)],
                                s_sh.at[sidx], add=True)
            plsc.subcore_barrier()
            pltpu.sync_copy(s_sh.at[pl.ds(z0, share)],
                            out_hbm.at[pl.ds(z0, share)])
            plsc.subcore_barrier()

        @pl.when(cid == 0)
        def _():
            run_pass(ud, sd, dd, 0, E_G, osd)
            run_pass(ue, sek, dek, 0, E_EK // 2, osea)

        @pl.when(cid == 1)
        def _():
            run_pass(uu, su, du, 0, E_G, osu)
            run_pass(ue, sek, dek, E_EK // 2, E_EK, oseb)

    return sc_sums


def _fuse_kn_body(kn_ref, dz_ref, ds_ref, uz_ref, us_ref,
                  w1_ref, b1_ref, w2_ref, b2_ref, out_ref):
    kn = kn_ref[...]
    bk = dz_ref[...] / (ds_ref[...] + 1e-9)
    ck = uz_ref[...] / (us_ref[...] + 1e-9)
    w1 = w1_ref[...]
    w2 = w2_ref[...]
    s1 = (jnp.dot(kn, w1[:, :D].T, preferred_element_type=_f32)
          + jnp.dot(bk, w1[:, D:].T, preferred_element_type=_f32)
          + b1_ref[...])
    s2 = (jnp.dot(kn, w2[:, :D].T, preferred_element_type=_f32)
          + jnp.dot(ck, w2[:, D:].T, preferred_element_type=_f32)
          + b2_ref[...])
    m = jnp.maximum(s1, s2)
    e1 = jnp.exp(s1 - m)
    e2 = jnp.exp(s2 - m)
    out_ref[...] = kn + (e1 * bk + e2 * ck) / (e1 + e2)


_fuse_kn = pl.pallas_call(
    _fuse_kn_body,
    out_shape=jax.ShapeDtypeStruct((NK, D), _f32),
)


def _fuse_ex_body(ex_ref, bz_ref, bs_ref, w_ref, b_ref, out_ref):
    ex = ex_ref[...]
    be = bz_ref[...] / (bs_ref[...] + 1e-9)
    w = w_ref[...]
    se = (jnp.dot(ex, w[:, :D].T, preferred_element_type=_f32)
          + jnp.dot(be, w[:, D:].T, preferred_element_type=_f32)
          + b_ref[...])
    out_ref[...] = ex + se * be


_EXB = 10000

_fuse_ex = pl.pallas_call(
    _fuse_ex_body,
    grid=(NE // _EXB,),
    in_specs=[
        pl.BlockSpec((_EXB, D), lambda i: (i, 0)),
        pl.BlockSpec((_EXB, D), lambda i: (i, 0)),
        pl.BlockSpec((_EXB, 1), lambda i: (i, 0)),
        pl.BlockSpec((1, 2 * D), lambda i: (0, 0)),
        pl.BlockSpec((1, 1), lambda i: (0, 0)),
    ],
    out_specs=pl.BlockSpec((_EXB, D), lambda i: (i, 0)),
    out_shape=jax.ShapeDtypeStruct((NE, D), _f32),
)


def kernel(exer_emb, kn_emb, ei_directed, ei_undirected, ei_ek,
           W_dir, A_dir, W_und, A_und, W_ek, A_ek,
           Wk1, bk1, Wk2, bk2, We1, be1):
    yd, yu, ye, ud, uu, ue = _prep(kn_emb, W_dir, W_und, W_ek,
                                   A_dir, A_und, A_ek)
    zz = jnp.zeros((TBLP, D), _f32)
    sd, dd = ei_directed[0], ei_directed[1]
    su, du = ei_undirected[0], ei_undirected[1]
    sek, dek = ei_ek[0] - NE, ei_ek[1]

    od, ou, oe = _build_sc_rows()(yd, yu, ye, sd, dd, su, du, sek, dek, zz)
    osd, osu, osea, oseb = _build_sc_sums()(
        ud.reshape(NK), uu.reshape(NK), ue.reshape(NK),
        sd, dd, su, du, sek, dek, zz)

    s_d = osd.reshape(-1)[:NK, None]
    s_u = osu.reshape(-1)[:NK, None]
    s_e = (osea + oseb).reshape(-1)[:NE, None]

    kn_out = _fuse_kn(kn_emb, od, s_d, ou, s_u,
                      Wk1, bk1.reshape(1, 1), Wk2, bk2.reshape(1, 1))
    ex_out = _fuse_ex(exer_emb, oe[:NE], s_e, We1, be1.reshape(1, 1))
    return ex_out, kn_out


# speedup vs baseline: 15.2066x; 1.4320x over previous
import functools

import jax
import jax.numpy as jnp
from jax import lax
from jax.experimental import pallas as pl
from jax.experimental.pallas import tpu as pltpu
from jax.experimental.pallas import tpu_sc as plsc

NK = 10000
NE = 50000
D = 128
CH = 12504
TRASH0 = CH
TBLP = 12544
BLK = 80
NSUB = 16
E_G = 320000
E_EK = 400000
NB_G = E_G // BLK
NB_EK = E_EK // BLK

SBLK = 320
SRW = 512

_f32 = jnp.float32


def _prep_body(kn_ref, w_ref, a_ref, y_ref, u_ref):
    kn = kn_ref[...]
    w = w_ref[...]
    z = jnp.dot(kn, w.T, preferred_element_type=_f32,
                precision=lax.Precision.HIGHEST)
    a1 = a_ref[...][:, :D]
    p = jnp.dot(z, a1.T, preferred_element_type=_f32,
                precision=lax.Precision.HIGHEST)
    u = jnp.exp(p - jnp.max(p))
    y_ref[...] = z * u
    u_ref[...] = u


_prep_one = pl.pallas_call(
    _prep_body,
    out_shape=[jax.ShapeDtypeStruct((NK, D), _f32),
               jax.ShapeDtypeStruct((NK, 1), _f32)],
)


def _prep(kn, wd, wu, we, ad, au, ae):
    yd, ud = _prep_one(kn, wd, ad)
    yu, uu = _prep_one(kn, wu, au)
    ye, ue = _prep_one(kn, we, ae)
    return yd, yu, ye, ud, uu, ue


@functools.lru_cache(maxsize=1)
def _build_sc_rows():
    mesh = plsc.VectorSubcoreMesh(core_axis_name="c", subcore_axis_name="s")

    @functools.partial(
        pl.kernel,
        out_type=[
            jax.ShapeDtypeStruct((NK, D), _f32),
            jax.ShapeDtypeStruct((NK, D), _f32),
            jax.ShapeDtypeStruct((4 * CH, D), _f32),
        ],
        mesh=mesh,
        compiler_params=pltpu.CompilerParams(needs_layout_passes=False),
        scratch_types=[
            pltpu.VMEM_SHARED((TBLP, D), _f32),
            pltpu.VMEM((2 * BLK,), jnp.int32),
            pltpu.VMEM((BLK,), jnp.int32),
            pltpu.VMEM((BLK, D), _f32),
            pltpu.SemaphoreType.DMA,
            pltpu.SemaphoreType.DMA,
            pltpu.VMEM((2 * BLK,), jnp.int32),
            pltpu.VMEM((BLK,), jnp.int32),
            pltpu.VMEM((BLK, D), _f32),
            pltpu.SemaphoreType.DMA,
            pltpu.SemaphoreType.DMA,
        ],
    )
    def sc_rows(yd, yu, ye, ed, eu, ee, zz,
                od, ou, oe,
                acc, iba, ida, rwa, sga, ssa, ibb, idb, rwb, sgb, ssb):
        sid = lax.axis_index("s")
        cid = lax.axis_index("c")
        bufs = ((iba, ida, rwa, sga, ssa), (ibb, idb, rwb, sgb, ssb))

        def run_pass(y_hbm, e_hbm, nblocks, base, chunked,
                     out_hbm, out_off, out_rows):
            zshare = TBLP // NSUB
            z0 = sid * zshare
            pltpu.sync_copy(zz.at[pl.ds(z0, zshare)],
                            acc.at[pl.ds(z0, zshare)])
            plsc.subcore_barrier()

            nloop = -(-nblocks // NSUB)
            npair = -(-nloop // 2)

            @pl.loop(0, npair)
            def _(j):
                for p in range(2):
                    ib, idr, rws, sg, ss = bufs[p]
                    b = (2 * j + p) * NSUB + sid

                    @pl.when((b >= 2 * NSUB) & (b < nblocks))
                    def _():
                        pltpu.make_async_copy(rws, acc.at[idr], ss).wait()

                    @pl.when(b < nblocks)
                    def _():
                        off = 2 * BLK * b
                        pltpu.sync_copy(e_hbm.at[pl.ds(off, 2 * BLK)], ib)
                        for k in range(BLK // 16):
                            d16 = ib[pl.ds(BLK + k * 16, 16)]
                            if chunked:
                                t = d16 - base
                                ok = (t >= 0) & (t < CH)
                                d16 = jnp.where(ok, t, TRASH0 + (d16 & 15))
                            idr[pl.ds(k * 16, 16)] = d16
                        pltpu.async_copy(
                            y_hbm.at[ib.at[pl.ds(0, BLK)]], rws, sg)

                for p in range(2):
                    ib, idr, rws, sg, ss = bufs[p]
                    b = (2 * j + p) * NSUB + sid

                    @pl.when(b < nblocks)
                    def _():
                        pltpu.make_async_copy(
                            y_hbm.at[ib.at[pl.ds(0, BLK)]], rws, sg).wait()
                        pltpu.async_copy(rws, acc.at[idr], ss, add=True)

            for p in range(2):
                ib, idr, rws, sg, ss = bufs[p]
                pltpu.make_async_copy(rws, acc.at[idr], ss).wait()

            plsc.subcore_barrier()
            lo = (out_rows // NSUB) & ~7
            rem = out_rows - (NSUB - 1) * lo

            @pl.when(sid < NSUB - 1)
            def _():
                o0 = sid * lo
                pltpu.sync_copy(acc.at[pl.ds(o0, lo)],
                                out_hbm.at[pl.ds(out_off + o0, lo)])

            @pl.when(sid == NSUB - 1)
            def _():
                o0 = (NSUB - 1) * lo
                pltpu.sync_copy(acc.at[pl.ds(o0, rem)],
                                out_hbm.at[pl.ds(out_off + o0, rem)])

            plsc.subcore_barrier()

        @pl.when(cid == 0)
        def _():
            run_pass(yd, ed, NB_G, 0, False, od, 0, NK)
            run_pass(ye, ee, NB_EK, 0, True, oe, 0, CH)
            run_pass(ye, ee, NB_EK, CH, True, oe, CH, CH)

        @pl.when(cid == 1)
        def _():
            run_pass(yu, eu, NB_G, 0, False, ou, 0, NK)
            run_pass(ye, ee, NB_EK, 2 * CH, True, oe, 2 * CH, CH)
            run_pass(ye, ee, NB_EK, 3 * CH, True, oe, 3 * CH, CH)

    return sc_rows


@functools.lru_cache(maxsize=1)
def _build_sc_sums():
    mesh = plsc.VectorSubcoreMesh(core_axis_name="c", subcore_axis_name="s")

    @functools.partial(
        pl.kernel,
        out_type=[
            jax.ShapeDtypeStruct((SRW, D), _f32),
            jax.ShapeDtypeStruct((SRW, D), _f32),
            jax.ShapeDtypeStruct((SRW, D), _f32),
            jax.ShapeDtypeStruct((SRW, D), _f32),
        ],
        mesh=mesh,
        compiler_params=pltpu.CompilerParams(needs_layout_passes=False),
        scratch_types=[
            pltpu.VMEM_SHARED((SRW, D), _f32),
            pltpu.VMEM((SRW, D), _f32),
            pltpu.VMEM((NK,), _f32),
            pltpu.VMEM((2 * SBLK,), jnp.int32),
            pltpu.VMEM((128,), jnp.int32),
        ],
    )
    def sc_sums(ud, uu, ue, ed, eu, ee, zz,
                osd, osu, osea, oseb,
                s_sh, s_loc, uv_t, ibuf, sidx):
        sid = lax.axis_index("s")
        cid = lax.axis_index("c")

        def run_pass(u_hbm, e_hbm, e_lo, e_hi, out_hbm):
            pltpu.sync_copy(zz.at[pl.ds(0, SRW)], s_loc)
            share = SRW // NSUB
            z0 = sid * share
            pltpu.sync_copy(zz.at[pl.ds(0, share)],
                            s_sh.at[pl.ds(z0, share)])
            pltpu.sync_copy(u_hbm, uv_t)
            plsc.subcore_barrier()

            nblocks = (e_hi - e_lo) // SBLK
            nloop = -(-nblocks // NSUB)

            @pl.loop(0, nloop)
            def _(i):
                b = i * NSUB + sid

                @pl.when(b < nblocks)
                def _():
                    off = 2 * e_lo + b * (2 * SBLK)
                    pltpu.sync_copy(e_hbm.at[pl.ds(off, 2 * SBLK)], ibuf)
                    for g in range(SBLK // BLK):
                        for k in range(BLK // 16):
                            s16 = ibuf[pl.ds(g * 2 * BLK + k * 16, 16)]
                            d16 = ibuf[pl.ds(g * 2 * BLK + BLK + k * 16, 16)]
                            uv = plsc.load_gather(uv_t, [s16])
                            plsc.addupdate_scatter(
                                s_loc,
                                [lax.shift_right_logical(d16, 7), d16 & 127],
                                uv)

            for seg in range(SRW // 128):
                for jj in range(8):
                    sidx[pl.ds(jj * 16, 16)] = (
                        lax.iota(jnp.int32, 16) + (seg * 128 + jj * 16))
                pltpu.sync_copy(s_loc.at[pl.ds(seg * 128, 128)],
                                s_sh.at[sidx], add=True)
            plsc.subcore_barrier()
            pltpu.sync_copy(s_sh.at[pl.ds(z0, share)],
                            out_hbm.at[pl.ds(z0, share)])
            plsc.subcore_barrier()

        @pl.when(cid == 0)
        def _():
            run_pass(ud, ed, 0, E_G, osd)
            run_pass(ue, ee, 0, E_EK // 2, osea)

        @pl.when(cid == 1)
        def _():
            run_pass(uu, eu, 0, E_G, osu)
            run_pass(ue, ee, E_EK // 2, E_EK, oseb)

    return sc_sums


def _fuse_kn_body(kn_ref, dz_ref, ds_ref, uz_ref, us_ref,
                  w1_ref, b1_ref, w2_ref, b2_ref, out_ref):
    kn = kn_ref[...]
    bk = dz_ref[...] / (ds_ref[...] + 1e-9)
    ck = uz_ref[...] / (us_ref[...] + 1e-9)
    w1 = w1_ref[...]
    w2 = w2_ref[...]
    s1 = (jnp.dot(kn, w1[:, :D].T, preferred_element_type=_f32)
          + jnp.dot(bk, w1[:, D:].T, preferred_element_type=_f32)
          + b1_ref[...])
    s2 = (jnp.dot(kn, w2[:, :D].T, preferred_element_type=_f32)
          + jnp.dot(ck, w2[:, D:].T, preferred_element_type=_f32)
          + b2_ref[...])
    m = jnp.maximum(s1, s2)
    e1 = jnp.exp(s1 - m)
    e2 = jnp.exp(s2 - m)
    out_ref[...] = kn + (e1 * bk + e2 * ck) / (e1 + e2)


_fuse_kn = pl.pallas_call(
    _fuse_kn_body,
    out_shape=jax.ShapeDtypeStruct((NK, D), _f32),
)


def _fuse_ex_body(ex_ref, bz_ref, bs_ref, w_ref, b_ref, out_ref):
    ex = ex_ref[...]
    be = bz_ref[...] / (bs_ref[...] + 1e-9)
    w = w_ref[...]
    se = (jnp.dot(ex, w[:, :D].T, preferred_element_type=_f32)
          + jnp.dot(be, w[:, D:].T, preferred_element_type=_f32)
          + b_ref[...])
    out_ref[...] = ex + se * be


_EXB = 10000

_fuse_ex = pl.pallas_call(
    _fuse_ex_body,
    grid=(NE // _EXB,),
    in_specs=[
        pl.BlockSpec((_EXB, D), lambda i: (i, 0)),
        pl.BlockSpec((_EXB, D), lambda i: (i, 0)),
        pl.BlockSpec((_EXB, 1), lambda i: (i, 0)),
        pl.BlockSpec((1, 2 * D), lambda i: (0, 0)),
        pl.BlockSpec((1, 1), lambda i: (0, 0)),
    ],
    out_specs=pl.BlockSpec((_EXB, D), lambda i: (i, 0)),
    out_shape=jax.ShapeDtypeStruct((NE, D), _f32),
)


def kernel(exer_emb, kn_emb, ei_directed, ei_undirected, ei_ek,
           W_dir, A_dir, W_und, A_und, W_ek, A_ek,
           Wk1, bk1, Wk2, bk2, We1, be1):
    yd, yu, ye, ud, uu, ue = _prep(kn_emb, W_dir, W_und, W_ek,
                                   A_dir, A_und, A_ek)
    zz = jnp.zeros((TBLP, D), _f32)

    def pack(s, d):
        return jnp.concatenate(
            [s.reshape(-1, BLK), d.reshape(-1, BLK)], axis=1).reshape(-1)

    ed = pack(ei_directed[0], ei_directed[1])
    eu = pack(ei_undirected[0], ei_undirected[1])
    ee = pack(ei_ek[0] - NE, ei_ek[1])

    od, ou, oe = _build_sc_rows()(yd, yu, ye, ed, eu, ee, zz)
    osd, osu, osea, oseb = _build_sc_sums()(
        ud.reshape(NK), uu.reshape(NK), ue.reshape(NK), ed, eu, ee, zz)

    s_d = osd.reshape(-1)[:NK, None]
    s_u = osu.reshape(-1)[:NK, None]
    s_e = (osea + oseb).reshape(-1)[:NE, None]

    kn_out = _fuse_kn(kn_emb, od, s_d, ou, s_u,
                      Wk1, bk1.reshape(1, 1), Wk2, bk2.reshape(1, 1))
    ex_out = _fuse_ex(exer_emb, oe[:NE], s_e, We1, be1.reshape(1, 1))
    return ex_out, kn_out
